# Initial kernel scaffold; baseline (speedup 1.0000x reference)
import jax
import jax.numpy as jnp
from jax.experimental import pallas as pl


def kernel(x, edge_index, src, dst, Wl0, Wr0, bs0, Wl1, Wr1, bs1, Wl2, Wr2, bs2, Wl3, Wr3, bs3, fcw0, fcb0, fcw1, fcb1, fcw2, fcb2, fcw3, fcb3):
    raise NotImplementedError("write your pallas kernel here")


# speedup vs baseline: 3.4687x; 3.4687x over previous
import functools

import jax
import jax.numpy as jnp
from jax import lax
from jax.experimental import pallas as pl
from jax.experimental.pallas import tpu as pltpu
from jax.experimental.pallas import tpu_sc as plsc

N = 10000
D = 128
E = 320000
NC = 2
NS = 16
NW = NC * NS
CH = 128
CPT = 79
CHUNKS = NW * CPT
E_PAD = CHUNKS * CH
NPAD = 10112
RPT = NPAD // NS
DUMMY_DST = N + 8

_mesh = plsc.VectorSubcoreMesh(core_axis_name="c", subcore_axis_name="s",
                               num_cores=NC, num_subcores=NS)


@functools.partial(
    pl.kernel,
    out_type=jax.ShapeDtypeStruct((NC, NPAD, D), jnp.float32),
    mesh=_mesh,
    scratch_types=[
        pltpu.VMEM_SHARED((NPAD, D), jnp.float32),
        pltpu.VMEM((40, CH), jnp.int32),
        pltpu.VMEM((40, CH), jnp.int32),
        pltpu.VMEM((CH, D), jnp.float32),
        pltpu.VMEM((CH, D), jnp.float32),
        pltpu.SemaphoreType.DMA,
        pltpu.SemaphoreType.DMA,
        pltpu.SemaphoreType.DMA,
        pltpu.SemaphoreType.DMA,
    ],
)
def _agg(h_hbm, srcs_hbm, dsts_hbm, zeros_hbm, out_hbm,
         acc_sh, src_v, dst_v, buf_a, buf_b, gsa, gsb, ssa, ssb):
    cid = lax.axis_index("c")
    sid = lax.axis_index("s")
    wid = sid * NC + cid
    pltpu.sync_copy(zeros_hbm, acc_sh.at[pl.ds(sid * RPT, RPT)])
    plsc.subcore_barrier()
    HSTG = 40
    for st, nch in ((0, HSTG), (HSTG, CPT - HSTG)):
        pltpu.sync_copy(srcs_hbm.at[wid, pl.ds(st, nch)],
                        src_v.at[pl.ds(0, nch)])
        pltpu.sync_copy(dsts_hbm.at[wid, pl.ds(st, nch)],
                        dst_v.at[pl.ds(0, nch)])

        NI = nch // 2
        pltpu.async_copy(h_hbm.at[src_v.at[0]], buf_a, gsa)

        @pl.loop(0, NI)
        def _pair(i):
            j0 = i * 2
            j1 = j0 + 1

            @pl.when(i > 0)
            def _drain_sb():
                pltpu.make_async_copy(buf_b, acc_sh.at[dst_v.at[j0 - 1]],
                                      ssb).wait()

            pltpu.make_async_copy(h_hbm.at[src_v.at[j0]], buf_a, gsa).wait()
            sa = pltpu.async_copy(buf_a, acc_sh.at[dst_v.at[j0]], ssa,
                                  add=True)
            gb = pltpu.async_copy(h_hbm.at[src_v.at[j1]], buf_b, gsb)
            sa.wait()
            gb.wait()
            pltpu.async_copy(buf_b, acc_sh.at[dst_v.at[j1]], ssb, add=True)

            @pl.when(i + 1 < NI)
            def _prefetch_ga():
                pltpu.async_copy(h_hbm.at[src_v.at[j0 + 2]], buf_a, gsa)

        pltpu.make_async_copy(buf_b, acc_sh.at[dst_v.at[NI * 2 - 1]],
                              ssb).wait()
        if nch % 2:
            j = nch - 1
            pltpu.async_copy(h_hbm.at[src_v.at[j]], buf_a, gsa).wait()
            pltpu.sync_copy(buf_a, acc_sh.at[dst_v.at[j]], add=True)

    plsc.subcore_barrier()
    pltpu.sync_copy(acc_sh.at[pl.ds(sid * RPT, RPT)],
                    out_hbm.at[cid, pl.ds(sid * RPT, RPT)])


@functools.partial(
    pl.kernel,
    out_type=jax.ShapeDtypeStruct((NC, NPAD, D), jnp.float32),
    mesh=_mesh,
    scratch_types=[
        pltpu.VMEM_SHARED((NPAD, D), jnp.float32),
        pltpu.VMEM((CPT, CH), jnp.int32),
        pltpu.VMEM((CH, D), jnp.float32),
        pltpu.SemaphoreType.DMA,
    ],
)
def _cnt(dsts_hbm, zeros_hbm, ones_hbm, out_hbm, cnt_sh, dst_v, ones_v, ssem):
    cid = lax.axis_index("c")
    sid = lax.axis_index("s")
    wid = sid * NC + cid
    pltpu.sync_copy(zeros_hbm, cnt_sh.at[pl.ds(sid * RPT, RPT)])
    pltpu.sync_copy(dsts_hbm.at[wid], dst_v)
    pltpu.sync_copy(ones_hbm, ones_v)
    plsc.subcore_barrier()

    @pl.loop(0, CPT)
    def _edges(j):
        pltpu.sync_copy(ones_v, cnt_sh.at[dst_v.at[j]], add=True)

    plsc.subcore_barrier()
    pltpu.sync_copy(cnt_sh.at[pl.ds(sid * RPT, RPT)],
                    out_hbm.at[cid, pl.ds(sid * RPT, RPT)])


@functools.partial(
    pl.kernel,
    out_type=jax.ShapeDtypeStruct((E_PAD, D), jnp.float32),
    mesh=_mesh,
    scratch_types=[
        pltpu.VMEM((CPT, CH), jnp.int32),
        pltpu.VMEM((CPT, CH), jnp.int32),
        pltpu.VMEM((CH, D), jnp.float32),
        pltpu.VMEM((CH, D), jnp.float32),
        pltpu.VMEM((CH, D), jnp.float32),
        pltpu.SemaphoreType.DMA,
        pltpu.SemaphoreType.DMA,
        pltpu.SemaphoreType.DMA,
        pltpu.SemaphoreType.DMA,
    ],
)
def _pairsum(u_hbm, v_hbm, srcs_hbm, dsts_hbm, out_hbm,
             src_v, dst_v, a0_v, a1_v, b_v, sem_a, sem_b, os0, os1):
    cid = lax.axis_index("c")
    sid = lax.axis_index("s")
    wid = sid * NC + cid
    base_c = wid * CPT
    pltpu.sync_copy(srcs_hbm.at[wid], src_v)
    pltpu.sync_copy(dsts_hbm.at[wid], dst_v)

    def _add(a_v):
        @pl.loop(0, CH)
        def _rows(r):
            for k in range(D // 16):
                sl = pl.ds(k * 16, 16)
                a_v[r, sl] = a_v[r, sl] + b_v[r, sl]

    def _rows_out(j):
        return pl.ds((base_c + j) * CH, CH)

    @pl.loop(0, CPT // 2)
    def _pair(i):
        j0 = i * 2

        ca = pltpu.async_copy(u_hbm.at[src_v.at[j0]], a0_v, sem_a)
        cb = pltpu.async_copy(v_hbm.at[dst_v.at[j0]], b_v, sem_b)
        ca.wait()
        cb.wait()

        @pl.when(i > 0)
        def _drain_prev_odd():
            pltpu.make_async_copy(a1_v, out_hbm.at[_rows_out(j0 - 1)],
                                  os1).wait()

        _add(a0_v)
        pltpu.async_copy(a0_v, out_hbm.at[_rows_out(j0)], os0)

        ca1 = pltpu.async_copy(u_hbm.at[src_v.at[j0 + 1]], a1_v, sem_a)
        cb1 = pltpu.async_copy(v_hbm.at[dst_v.at[j0 + 1]], b_v, sem_b)
        ca1.wait()
        cb1.wait()
        pltpu.make_async_copy(a0_v, out_hbm.at[_rows_out(j0)], os0).wait()
        _add(a1_v)
        pltpu.async_copy(a1_v, out_hbm.at[_rows_out(j0 + 1)], os1)

    pltpu.make_async_copy(a1_v, out_hbm.at[_rows_out(CPT // 2 * 2 - 1)],
                          os1).wait()
    j = CPT - 1
    pltpu.async_copy(u_hbm.at[src_v.at[j]], a0_v, sem_a).wait()
    pltpu.async_copy(v_hbm.at[dst_v.at[j]], b_v, sem_b).wait()
    _add(a0_v)
    pltpu.sync_copy(a0_v, out_hbm.at[_rows_out(j)])


_NB = 10
_RB = N // _NB


def _norm_body(x_ref, o_ref):
    x = x_ref[...]
    n = jnp.sqrt(jnp.sum(x * x, axis=1, keepdims=True))
    o_ref[...] = x / jnp.maximum(n, 1e-12)


_normalize = pl.pallas_call(
    _norm_body,
    grid=(_NB,),
    in_specs=[pl.BlockSpec((_RB, D), lambda i: (i, 0))],
    out_specs=pl.BlockSpec((_RB, D), lambda i: (i, 0)),
    out_shape=jax.ShapeDtypeStruct((N, D), jnp.float32),
)


def _mean(aggA, aggB, cntA, cntB):
    s = aggA[0] + aggB[0]
    c = cntA[0, :, 0:1] + cntB[0, :, 0:1]
    return s / jnp.maximum(c, 1.0)


def _dense_body(aggA, aggB, cntA, cntB, h_ref, wl, wr, b, o_ref):
    mean = _mean(aggA, aggB, cntA, cntB)
    o = (jnp.dot(mean, wl[...], preferred_element_type=jnp.float32)
         + jnp.dot(h_ref[...], wr[...], preferred_element_type=jnp.float32)
         + b[...])
    o_ref[...] = jnp.maximum(o, 0.0)


def _dense3_body(aggA, aggB, cntA, cntB, h_ref, wl, wr, b, w0a, w0b, b0,
                 u_ref, v_ref):
    mean = _mean(aggA, aggB, cntA, cntB)
    z = (jnp.dot(mean, wl[...], preferred_element_type=jnp.float32)
         + jnp.dot(h_ref[...], wr[...], preferred_element_type=jnp.float32)
         + b[...])
    z = jnp.maximum(z, 0.0)
    u_ref[...] = jnp.dot(z, w0a[...], preferred_element_type=jnp.float32) + b0[...]
    v_ref[...] = jnp.dot(z, w0b[...], preferred_element_type=jnp.float32)


_agg_spec = pl.BlockSpec((1, _RB, D), lambda i: (0, i, 0))
_agg_spec2 = pl.BlockSpec((1, _RB, D), lambda i: (1, i, 0))
_cnt_spec = pl.BlockSpec((1, _RB, D), lambda i: (0, i, 0))
_cnt_spec2 = pl.BlockSpec((1, _RB, D), lambda i: (1, i, 0))
_row_spec = pl.BlockSpec((_RB, D), lambda i: (i, 0))


def _full(shape):
    return pl.BlockSpec(shape, lambda i: tuple(0 for _ in shape))


_dense = pl.pallas_call(
    _dense_body,
    grid=(_NB,),
    in_specs=[_agg_spec, _agg_spec2, _cnt_spec, _cnt_spec2, _row_spec,
              _full((D, D)), _full((D, D)), _full((1, D))],
    out_specs=_row_spec,
    out_shape=jax.ShapeDtypeStruct((N, D), jnp.float32),
)

_dense3 = pl.pallas_call(
    _dense3_body,
    grid=(_NB,),
    in_specs=[_agg_spec, _agg_spec2, _cnt_spec, _cnt_spec2, _row_spec,
              _full((D, D)), _full((D, D)), _full((1, D)),
              _full((D, D)), _full((D, D)), _full((1, D))],
    out_specs=[_row_spec, _row_spec],
    out_shape=[jax.ShapeDtypeStruct((N, D), jnp.float32),
               jax.ShapeDtypeStruct((N, D), jnp.float32)],
)


def _leaky(x):
    return jnp.maximum(x, 0.01 * x)


def _mlp_body(p_ref, w1, b1, w2, b2, w3, b3, o_ref):
    p = _leaky(p_ref[...])
    a = _leaky(jnp.dot(p, w1[...], preferred_element_type=jnp.float32) + b1[...])
    a = _leaky(jnp.dot(a, w2[...], preferred_element_type=jnp.float32) + b2[...])
    o_ref[...] = jnp.dot(a, w3[...], preferred_element_type=jnp.float32) + b3[...]


_EB = 1000

_mlp = pl.pallas_call(
    _mlp_body,
    grid=(E // _EB,),
    in_specs=[pl.BlockSpec((_EB, D), lambda i: (i, 0)),
              _full((D, 64)), _full((1, 64)),
              _full((64, 16)), _full((1, 16)),
              _full((16, 1)), _full((1, 1))],
    out_specs=pl.BlockSpec((_EB, 1), lambda i: (i, 0)),
    out_shape=jax.ShapeDtypeStruct((E, 1), jnp.float32),
)


def _pad_idx(a, fill):
    pad = jnp.full((E_PAD - E,), fill, jnp.int32)
    return jnp.concatenate([a.astype(jnp.int32), pad]).reshape(NW, CPT, CH)


def kernel(x, edge_index, src, dst,
           Wl0, Wr0, bs0, Wl1, Wr1, bs1, Wl2, Wr2, bs2, Wl3, Wr3, bs3,
           fcw0, fcb0, fcw1, fcb1, fcw2, fcb2, fcw3, fcb3):
    srcE = _pad_idx(edge_index[0], 0)
    dstE = _pad_idx(edge_index[1], DUMMY_DST)
    srcD = _pad_idx(src, 0)
    dstD = _pad_idx(dst, 0)
    zD = jnp.zeros((RPT, D), jnp.float32)

    h = _normalize(x)
    cnt = _cnt(dstE, zD, jnp.ones((CH, D), jnp.float32))

    Wls = (Wl0, Wl1, Wl2)
    Wrs = (Wr0, Wr1, Wr2)
    bss = (bs0, bs1, bs2)
    for i in range(3):
        agg = _agg(h, srcE, dstE, zD)
        h = _dense(agg, agg, cnt, cnt, h,
                   Wls[i], Wrs[i], bss[i].reshape(1, D))
    agg = _agg(h, srcE, dstE, zD)
    U, V = _dense3(agg, agg, cnt, cnt, h,
                   Wl3, Wr3, bs3.reshape(1, D),
                   fcw0[:D], fcw0[D:], fcb0.reshape(1, D))
    P = _pairsum(U, V, srcD, dstD)
    return _mlp(P, fcw1, fcb1.reshape(1, 64), fcw2, fcb2.reshape(1, 16),
                fcw3, fcb3.reshape(1, 1))
